# baseline (speedup 1.0000x reference)
import jax
import jax.numpy as jnp
from jax import lax
from jax.experimental import pallas as pl
from jax.experimental.pallas import tpu as pltpu
from jax.experimental.pallas import tpu_sc as plsc

D_ROW_K = 384
D_COL_K = 384
H_K = 32
W_K = 32
N_K = H_K * W_K
NW_K = 32
B_PER_W_K = N_K // NW_K


def _sc_body(row_hbm, col_hbm, idx_row_hbm, idx_col_hbm, out_hbm,
             idx_r_v, idx_c_v, rows_v, cols_v, sem_r, sem_c):
    wid = lax.axis_index("s") * 2 + lax.axis_index("c")
    base = wid * B_PER_W_K
    pltpu.sync_copy(idx_row_hbm.at[pl.ds(base, B_PER_W_K)], idx_r_v)
    pltpu.sync_copy(idx_col_hbm.at[pl.ds(base, B_PER_W_K)], idx_c_v)
    cr = pltpu.async_copy(row_hbm.at[idx_r_v], rows_v, sem_r)
    cc = pltpu.async_copy(col_hbm.at[idx_c_v], cols_v, sem_c)
    cr.wait()
    cc.wait()
    pltpu.sync_copy(rows_v, out_hbm.at[pl.ds(base, B_PER_W_K), pl.ds(0, D_ROW_K)])
    pltpu.sync_copy(cols_v, out_hbm.at[pl.ds(base, B_PER_W_K), pl.ds(D_ROW_K, D_COL_K)])


def kernel(h, w, row_table, col_table):
    n = jnp.arange(N_K, dtype=jnp.int32)
    hm1 = jnp.asarray(h, jnp.int32) - 1
    wm1 = jnp.asarray(w, jnp.int32) - 1
    idx_row = jnp.minimum(n // W_K, hm1)
    idx_col = jnp.minimum(n % W_K, wm1)
    k = pl.kernel(
        _sc_body,
        mesh=plsc.VectorSubcoreMesh(core_axis_name="c", subcore_axis_name="s"),
        out_type=jax.ShapeDtypeStruct((N_K, D_ROW_K + D_COL_K), jnp.float32),
        scratch_types=[
            pltpu.VMEM((B_PER_W_K,), jnp.int32),
            pltpu.VMEM((B_PER_W_K,), jnp.int32),
            pltpu.VMEM((B_PER_W_K, D_ROW_K), jnp.float32),
            pltpu.VMEM((B_PER_W_K, D_COL_K), jnp.float32),
            pltpu.SemaphoreType.DMA,
            pltpu.SemaphoreType.DMA,
        ],
    )
    return k(row_table, col_table, idx_row, idx_col)
